# baseline (speedup 1.0000x reference)
import functools

import jax
import jax.numpy as jnp
from jax import lax
from jax.experimental import pallas as pl
from jax.experimental.pallas import tpu as pltpu
from jax.experimental.pallas import tpu_sc as plsc

_K = 32
_N = 4096
_D = 32
_BT = 96
_ROW_BLK = 512
_LANES = 128
_NCAND = 512
_N_ITERS = 22
_T_BLK = 1

_N_WORKERS = 32
_RPW = _N // _N_WORKERS
_BATCH = 16
_QUAD = 4


def _reduce_kernel(x_ref, acc_ref):
    @pl.when(pl.program_id(0) == 0)
    def _init():
        acc_ref[...] = jnp.zeros_like(acc_ref)

    acc_ref[...] += jnp.sum(x_ref[...], axis=(0, 1))


def _select_kernel(xsb_ref, xs_ref, vals_ref, idx_ref):
    inv = 1.0 / _BT
    xs = xs_ref[...] * inv
    norm = jnp.sqrt(jnp.sum(xs * xs, axis=-1, keepdims=True))
    xn = xs / jnp.maximum(norm, 1e-8)
    xmb = xsb_ref[...] * inv
    normb = jnp.sqrt(jnp.sum(xmb * xmb, axis=-1, keepdims=True))
    xnb = xmb / jnp.maximum(normb, 1e-8)

    sim = jax.lax.dot_general(
        xnb, xn, (((1,), (1,)), ((), ())),
        preferred_element_type=jnp.float32,
    )

    neg = jnp.full((_ROW_BLK, _LANES), -3.0, jnp.float32)
    lane = jax.lax.broadcasted_iota(jnp.int32, (_ROW_BLK, _LANES), 1)
    zero_i = jnp.zeros((_ROW_BLK, _LANES), jnp.int32)
    m1, m2, m3, m4 = neg, neg, neg, neg
    i1, i2, i3, i4 = zero_i, zero_i, zero_i, zero_i
    for c in range(_N // _LANES):
        v = sim[:, c * _LANES:(c + 1) * _LANES]
        iv = lane + (c * _LANES)
        b = v > m1
        t = jnp.minimum(m1, v)
        ti = jnp.where(b, i1, iv)
        m1 = jnp.maximum(m1, v)
        i1 = jnp.where(b, iv, i1)
        v, iv = t, ti
        b = v > m2
        t = jnp.minimum(m2, v)
        ti = jnp.where(b, i2, iv)
        m2 = jnp.maximum(m2, v)
        i2 = jnp.where(b, iv, i2)
        v, iv = t, ti
        b = v > m3
        t = jnp.minimum(m3, v)
        ti = jnp.where(b, i3, iv)
        m3 = jnp.maximum(m3, v)
        i3 = jnp.where(b, iv, i3)
        v, iv = t, ti
        b = v > m4
        m4 = jnp.maximum(m4, v)
        i4 = jnp.where(b, iv, i4)
    cand = jnp.concatenate([m1, m2, m3, m4], axis=1)
    cidx = jnp.concatenate([i1, i2, i3, i4], axis=1)

    lo = jnp.full((_ROW_BLK, 1), -1.5, jnp.float32)
    hi = jnp.full((_ROW_BLK, 1), 1.5, jnp.float32)

    def body(_, carry):
        lo, hi = carry
        mid = (lo + hi) * 0.5
        cnt = jnp.sum((cand >= mid).astype(jnp.float32), axis=1, keepdims=True)
        ge = cnt >= _K
        return jnp.where(ge, mid, lo), jnp.where(ge, hi, mid)

    lo, hi = jax.lax.fori_loop(0, _N_ITERS, body, (lo, hi))

    vals = jnp.where(cand >= lo, cand, 0.0)
    r0 = pl.program_id(0) * _ROW_BLK
    rowv = jax.lax.broadcasted_iota(jnp.int32, (_ROW_BLK, _NCAND), 0) + r0
    vals = jnp.where(cidx == rowv, 1.0, vals)

    vals_ref[...] = vals.reshape(vals_ref.shape)
    idx_ref[...] = cidx.reshape(idx_ref.shape)


def _make_scatter_kernel():
    mesh = plsc.VectorSubcoreMesh(core_axis_name="c", subcore_axis_name="s")
    n_quads = _RPW // _QUAD

    @functools.partial(
        pl.kernel,
        out_type=jax.ShapeDtypeStruct((_N, _N), jnp.float32),
        mesh=mesh,
        scratch_types=(
            [pltpu.VMEM((_NCAND,), jnp.float32) for _ in range(2 * _QUAD)]
            + [pltpu.VMEM((_NCAND,), jnp.int32) for _ in range(2 * _QUAD)]
            + [pltpu.VMEM((_N,), jnp.float32) for _ in range(_QUAD)]
            + [pltpu.SemaphoreType.DMA, pltpu.SemaphoreType.DMA,
               pltpu.SemaphoreType.DMA]
        ),
        compiler_params=pltpu.CompilerParams(needs_layout_passes=False),
    )
    def scatter_kernel(vals_hbm, idx_hbm, out_hbm, *refs):
        vbufs = [refs[0:_QUAD], refs[_QUAD:2 * _QUAD]]
        ibufs = [refs[2 * _QUAD:3 * _QUAD], refs[3 * _QUAD:4 * _QUAD]]
        rowbufs = refs[4 * _QUAD:5 * _QUAD]
        in_sems = (refs[5 * _QUAD], refs[5 * _QUAD + 1])
        out_sem = refs[5 * _QUAD + 2]

        wid = lax.axis_index("s") * 2 + lax.axis_index("c")
        base = wid * _RPW
        zero16 = jnp.zeros((16,), jnp.float32)

        def zinit(i, _):
            for rb in rowbufs:
                rb[pl.ds(i * 16, 16)] = zero16
            return 0

        lax.fori_loop(0, _N // 16, zinit, 0)

        def fire_quad(q, grp):
            for s in range(_QUAD):
                r = base + q * _QUAD + s
                pltpu.async_copy(vals_hbm.at[r], vbufs[grp][s], in_sems[grp])
                pltpu.async_copy(idx_hbm.at[r], ibufs[grp][s], in_sems[grp])

        fire_quad(0, 0)

        def quad_body(q, _):
            grp = lax.rem(q, 2)

            @pl.when(q + 1 < n_quads)
            def _prefetch():
                def fire_dyn(grp_next):
                    for s in range(_QUAD):
                        r = base + (q + 1) * _QUAD + s
                        pltpu.async_copy(
                            vals_hbm.at[r], vbufs[grp_next][s],
                            in_sems[grp_next])
                        pltpu.async_copy(
                            idx_hbm.at[r], ibufs[grp_next][s],
                            in_sems[grp_next])

                lax.cond(grp == 0, lambda: fire_dyn(1), lambda: fire_dyn(0))

            def process(grp_i):
                for s in range(_QUAD):
                    r = base + q * _QUAD + s
                    pltpu.make_async_copy(
                        vals_hbm.at[r], vbufs[grp_i][s], in_sems[grp_i]).wait()
                    pltpu.make_async_copy(
                        idx_hbm.at[r], ibufs[grp_i][s], in_sems[grp_i]).wait()
                out_copies = []
                for s in range(_QUAD):
                    r = base + q * _QUAD + s
                    for c in range(_NCAND // 16):
                        v = vbufs[grp_i][s][pl.ds(c * 16, 16)]
                        ix = ibufs[grp_i][s][pl.ds(c * 16, 16)]
                        plsc.store_scatter(rowbufs[s], [ix], v)
                    out_copies.append(
                        pltpu.async_copy(rowbufs[s], out_hbm.at[r], out_sem))
                for cp in out_copies:
                    cp.wait()
                for s in range(_QUAD):
                    for c in range(_NCAND // 16):
                        ix = ibufs[grp_i][s][pl.ds(c * 16, 16)]
                        plsc.store_scatter(rowbufs[s], [ix], zero16)

            lax.cond(grp == 0, lambda: process(0), lambda: process(1))
            return 0

        lax.fori_loop(0, n_quads, quad_body, 0)

    return scatter_kernel


def kernel(x):
    B, T, N, D = x.shape

    xsum = pl.pallas_call(
        _reduce_kernel,
        grid=(T // _T_BLK,),
        in_specs=[pl.BlockSpec((B, _T_BLK, N, D), lambda i: (0, i, 0, 0))],
        out_specs=pl.BlockSpec((N, D), lambda i: (0, 0)),
        out_shape=jax.ShapeDtypeStruct((N, D), jnp.float32),
    )(x)

    vals, idx = pl.pallas_call(
        _select_kernel,
        grid=(N // _ROW_BLK,),
        in_specs=[
            pl.BlockSpec((_ROW_BLK, D), lambda i: (i, 0)),
            pl.BlockSpec((N, D), lambda i: (0, 0)),
        ],
        out_specs=[
            pl.BlockSpec((_ROW_BLK, _NCAND), lambda i: (i, 0)),
            pl.BlockSpec((_ROW_BLK, _NCAND), lambda i: (i, 0)),
        ],
        out_shape=[
            jax.ShapeDtypeStruct((_N, _NCAND), jnp.float32),
            jax.ShapeDtypeStruct((_N, _NCAND), jnp.int32),
        ],
    )(xsum, xsum)

    adj = _make_scatter_kernel()(vals, idx)
    return adj


# speedup vs baseline: 1.0213x; 1.0213x over previous
import functools

import jax
import jax.numpy as jnp
from jax import lax
from jax.experimental import pallas as pl
from jax.experimental.pallas import tpu as pltpu
from jax.experimental.pallas import tpu_sc as plsc

_K = 32
_N = 4096
_D = 32
_BT = 96
_ROW_BLK = 512
_LANES = 128
_NCAND = 512
_N_ITERS = 22
_T_BLK = 1

_N_WORKERS = 32
_RPW = _N // _N_WORKERS
_BATCH = 16
_QUAD = 4


def _reduce_kernel(x_ref, acc_ref):
    @pl.when(pl.program_id(0) == 0)
    def _init():
        acc_ref[...] = jnp.zeros_like(acc_ref)

    acc_ref[...] += jnp.sum(x_ref[...], axis=(0, 1))


def _select_kernel(xsb_ref, xs_ref, vals_ref, idx_ref):
    inv = 1.0 / _BT
    xs = xs_ref[...] * inv
    norm = jnp.sqrt(jnp.sum(xs * xs, axis=-1, keepdims=True))
    xn = xs / jnp.maximum(norm, 1e-8)
    xmb = xsb_ref[...] * inv
    normb = jnp.sqrt(jnp.sum(xmb * xmb, axis=-1, keepdims=True))
    xnb = xmb / jnp.maximum(normb, 1e-8)

    sim = jax.lax.dot_general(
        xnb, xn, (((1,), (1,)), ((), ())),
        preferred_element_type=jnp.float32,
    )

    neg = jnp.full((_ROW_BLK, _LANES), -3.0, jnp.float32)
    lane = jax.lax.broadcasted_iota(jnp.int32, (_ROW_BLK, _LANES), 1)
    m1, m2, m3, m4 = neg, neg, neg, neg
    for c in range(_N // _LANES):
        v = sim[:, c * _LANES:(c + 1) * _LANES]
        t = jnp.minimum(m1, v)
        m1 = jnp.maximum(m1, v)
        v = t
        t = jnp.minimum(m2, v)
        m2 = jnp.maximum(m2, v)
        v = t
        t = jnp.minimum(m3, v)
        m3 = jnp.maximum(m3, v)
        m4 = jnp.maximum(m4, t)
    zero_i = jnp.zeros((_ROW_BLK, _LANES), jnp.int32)
    i1, i2, i3, i4 = zero_i, zero_i, zero_i, zero_i
    for c in range(_N // _LANES):
        v = sim[:, c * _LANES:(c + 1) * _LANES]
        iv = lane + (c * _LANES)
        i1 = jnp.where(v == m1, iv, i1)
        i2 = jnp.where(v == m2, iv, i2)
        i3 = jnp.where(v == m3, iv, i3)
        i4 = jnp.where(v == m4, iv, i4)
    cand = jnp.concatenate([m1, m2, m3, m4], axis=1)
    cidx = jnp.concatenate([i1, i2, i3, i4], axis=1)

    lo = jnp.full((_ROW_BLK, 1), -1.5, jnp.float32)
    hi = jnp.full((_ROW_BLK, 1), 1.5, jnp.float32)

    def body(_, carry):
        lo, hi = carry
        mid = (lo + hi) * 0.5
        cnt = jnp.sum((cand >= mid).astype(jnp.float32), axis=1, keepdims=True)
        ge = cnt >= _K
        return jnp.where(ge, mid, lo), jnp.where(ge, hi, mid)

    lo, hi = jax.lax.fori_loop(0, _N_ITERS, body, (lo, hi))

    vals = jnp.where(cand >= lo, cand, 0.0)
    r0 = pl.program_id(0) * _ROW_BLK
    rowv = jax.lax.broadcasted_iota(jnp.int32, (_ROW_BLK, _NCAND), 0) + r0
    vals = jnp.where(cidx == rowv, 1.0, vals)

    vals_ref[...] = vals.reshape(vals_ref.shape)
    idx_ref[...] = cidx.reshape(idx_ref.shape)


def _make_scatter_kernel():
    mesh = plsc.VectorSubcoreMesh(core_axis_name="c", subcore_axis_name="s")
    n_quads = _RPW // _QUAD

    @functools.partial(
        pl.kernel,
        out_type=jax.ShapeDtypeStruct((_N, _N), jnp.float32),
        mesh=mesh,
        scratch_types=(
            [pltpu.VMEM((_NCAND,), jnp.float32) for _ in range(2 * _QUAD)]
            + [pltpu.VMEM((_NCAND,), jnp.int32) for _ in range(2 * _QUAD)]
            + [pltpu.VMEM((_N,), jnp.float32) for _ in range(_QUAD)]
            + [pltpu.SemaphoreType.DMA, pltpu.SemaphoreType.DMA,
               pltpu.SemaphoreType.DMA]
        ),
        compiler_params=pltpu.CompilerParams(needs_layout_passes=False),
    )
    def scatter_kernel(vals_hbm, idx_hbm, out_hbm, *refs):
        vbufs = [refs[0:_QUAD], refs[_QUAD:2 * _QUAD]]
        ibufs = [refs[2 * _QUAD:3 * _QUAD], refs[3 * _QUAD:4 * _QUAD]]
        rowbufs = refs[4 * _QUAD:5 * _QUAD]
        in_sems = (refs[5 * _QUAD], refs[5 * _QUAD + 1])
        out_sem = refs[5 * _QUAD + 2]

        wid = lax.axis_index("s") * 2 + lax.axis_index("c")
        base = wid * _RPW
        zero16 = jnp.zeros((16,), jnp.float32)

        def zinit(i, _):
            for rb in rowbufs:
                rb[pl.ds(i * 16, 16)] = zero16
            return 0

        lax.fori_loop(0, _N // 16, zinit, 0)

        def fire_quad(q, grp):
            for s in range(_QUAD):
                r = base + q * _QUAD + s
                pltpu.async_copy(vals_hbm.at[r], vbufs[grp][s], in_sems[grp])
                pltpu.async_copy(idx_hbm.at[r], ibufs[grp][s], in_sems[grp])

        fire_quad(0, 0)

        def quad_body(q, _):
            grp = lax.rem(q, 2)

            @pl.when(q + 1 < n_quads)
            def _prefetch():
                def fire_dyn(grp_next):
                    for s in range(_QUAD):
                        r = base + (q + 1) * _QUAD + s
                        pltpu.async_copy(
                            vals_hbm.at[r], vbufs[grp_next][s],
                            in_sems[grp_next])
                        pltpu.async_copy(
                            idx_hbm.at[r], ibufs[grp_next][s],
                            in_sems[grp_next])

                lax.cond(grp == 0, lambda: fire_dyn(1), lambda: fire_dyn(0))

            def process(grp_i):
                for s in range(_QUAD):
                    r = base + q * _QUAD + s
                    pltpu.make_async_copy(
                        vals_hbm.at[r], vbufs[grp_i][s], in_sems[grp_i]).wait()
                    pltpu.make_async_copy(
                        idx_hbm.at[r], ibufs[grp_i][s], in_sems[grp_i]).wait()
                out_copies = []
                for s in range(_QUAD):
                    r = base + q * _QUAD + s
                    for c in range(_NCAND // 16):
                        v = vbufs[grp_i][s][pl.ds(c * 16, 16)]
                        ix = ibufs[grp_i][s][pl.ds(c * 16, 16)]
                        plsc.store_scatter(rowbufs[s], [ix], v)
                    out_copies.append(
                        pltpu.async_copy(rowbufs[s], out_hbm.at[r], out_sem))
                for cp in out_copies:
                    cp.wait()
                for s in range(_QUAD):
                    for c in range(_NCAND // 16):
                        ix = ibufs[grp_i][s][pl.ds(c * 16, 16)]
                        plsc.store_scatter(rowbufs[s], [ix], zero16)

            lax.cond(grp == 0, lambda: process(0), lambda: process(1))
            return 0

        lax.fori_loop(0, n_quads, quad_body, 0)

    return scatter_kernel


def kernel(x):
    B, T, N, D = x.shape

    xsum = pl.pallas_call(
        _reduce_kernel,
        grid=(T // _T_BLK,),
        in_specs=[pl.BlockSpec((B, _T_BLK, N, D), lambda i: (0, i, 0, 0))],
        out_specs=pl.BlockSpec((N, D), lambda i: (0, 0)),
        out_shape=jax.ShapeDtypeStruct((N, D), jnp.float32),
    )(x)

    vals, idx = pl.pallas_call(
        _select_kernel,
        grid=(N // _ROW_BLK,),
        in_specs=[
            pl.BlockSpec((_ROW_BLK, D), lambda i: (i, 0)),
            pl.BlockSpec((N, D), lambda i: (0, 0)),
        ],
        out_specs=[
            pl.BlockSpec((_ROW_BLK, _NCAND), lambda i: (i, 0)),
            pl.BlockSpec((_ROW_BLK, _NCAND), lambda i: (i, 0)),
        ],
        out_shape=[
            jax.ShapeDtypeStruct((_N, _NCAND), jnp.float32),
            jax.ShapeDtypeStruct((_N, _NCAND), jnp.int32),
        ],
    )(xsum, xsum)

    adj = _make_scatter_kernel()(vals, idx)
    return adj
